# baseline (speedup 1.0000x reference)
import jax
import jax.numpy as jnp
from jax.experimental import pallas as pl
from jax.experimental.pallas import tpu as pltpu

_T = 2048
_D = 768
_H = 64
_E = 64
_EPG = 8


def _moe_kernel(x_ref, rin_ref, rl_ref, ge_ref, lin_ref, pes_ref,
                out_ref, idx_scr):
    s = pl.program_id(0)

    @pl.when(s == 0)
    def _router():
        logits = jax.lax.dot_general(
            rin_ref[...], rl_ref[...], (((1,), (0,)), ((), ())),
            preferred_element_type=jnp.float32)
        m = jnp.max(logits, axis=1, keepdims=True)
        ii = jax.lax.broadcasted_iota(jnp.int32, (_T, _E), 1)
        idx_scr[...] = jnp.min(jnp.where(logits == m, ii, _E),
                               axis=1, keepdims=True)

    x = x_ref[...]
    w = ge_ref[...].reshape(_EPG * 2 * _H, _D).astype(jnp.bfloat16)
    g = jax.lax.dot_general(x, w, (((1,), (1,)), ((), ())),
                            preferred_element_type=jnp.float32)
    idx = idx_scr[...]
    acts = []
    for e in range(_EPG):
        g0 = g[:, e * 2 * _H:e * 2 * _H + _H]
        g1 = g[:, e * 2 * _H + _H:(e + 1) * 2 * _H]
        sel = jnp.where(idx == s * _EPG + e, pes_ref[0, e, 0], 0.0)
        acts.append(jax.nn.gelu(g0) * g1 * sel)
    act = jnp.concatenate(acts, axis=1).astype(jnp.bfloat16)
    lw = lin_ref[...].reshape(_EPG * _H, _D).astype(jnp.bfloat16)
    out_e = jax.lax.dot_general(act, lw, (((1,), (0,)), ((), ())),
                                preferred_element_type=jnp.float32)

    @pl.when(s == 0)
    def _init():
        out_ref[...] = out_e

    @pl.when(s > 0)
    def _acc():
        out_ref[...] += out_e


def kernel(x, router_scale, router_logits, gating_einsum, linear, per_expert_scale):
    B, L, D = x.shape
    x32 = x.reshape(L, D).astype(jnp.float32)
    variance = jnp.mean(jnp.square(x32), axis=-1, keepdims=True)
    rin = x32 * jax.lax.rsqrt(variance + 1e-06)
    root = jax.lax.rsqrt(jnp.array(D, dtype=rin.dtype))
    rin = rin * root * router_scale.astype(rin.dtype)

    x_bf = x32.astype(jnp.bfloat16)
    rin_bf = rin.astype(jnp.bfloat16)
    rl_bf = router_logits.astype(jnp.bfloat16)
    pes3 = per_expert_scale.reshape(_E // _EPG, _EPG, 1)

    out = pl.pallas_call(
        _moe_kernel,
        grid=(_E // _EPG,),
        in_specs=[
            pl.BlockSpec((_T, _D), lambda s: (0, 0)),
            pl.BlockSpec((_T, _D), lambda s: (0, 0)),
            pl.BlockSpec((_D, _E), lambda s: (0, 0)),
            pl.BlockSpec((_EPG, 2, _H, _D), lambda s: (s, 0, 0, 0)),
            pl.BlockSpec((_EPG, _H, _D), lambda s: (s, 0, 0)),
            pl.BlockSpec((1, _EPG, 1), lambda s: (s, 0, 0)),
        ],
        out_specs=pl.BlockSpec((_T, _D), lambda s: (0, 0)),
        out_shape=jax.ShapeDtypeStruct((_T, _D), jnp.float32),
        scratch_shapes=[pltpu.VMEM((_T, 1), jnp.int32)],
    )(x_bf, rin_bf, rl_bf, gating_einsum, linear, pes3)
    return out.reshape(B, L, D)
